# baseline (speedup 1.0000x reference)
import jax
import jax.numpy as jnp
from jax import lax
from jax.experimental import pallas as pl
from jax.experimental.pallas import tpu as pltpu
from jax.experimental.pallas import tpu_sc as plsc

N_NODES = 10000
IN_CH = 128
HID = 128
NUM_CLASSES = 10
NUM_GRAPHS = 128

NP = 10240
E = 320000
CH = 128
NCH = 80
NTILES = 32
EP = NTILES * NCH * CH
RPT = NP // 16
BLK = 1024
NBLK = NP // BLK


def _deg_body(dst2, ones_hbm, zeros1_hbm, out, didx, ones_v, acc1):
    c = lax.axis_index("c")
    s = lax.axis_index("s")
    tid = s * 2 + c
    pltpu.sync_copy(dst2.at[pl.ds(tid * NCH, NCH)], didx)
    pltpu.sync_copy(ones_hbm, ones_v)
    pltpu.sync_copy(zeros1_hbm, acc1.at[pl.ds(s * RPT, RPT)])
    plsc.subcore_barrier()

    def body(k, carry):
        pltpu.sync_copy(ones_v, acc1.at[didx.at[k]], add=True)
        return carry

    lax.fori_loop(0, NCH, body, 0)
    plsc.subcore_barrier()
    pltpu.sync_copy(acc1.at[pl.ds(s * RPT, RPT)], out.at[c, pl.ds(s * RPT, RPT)])


def _sc_calls():
    mesh = plsc.VectorSubcoreMesh(core_axis_name="c", subcore_axis_name="s",
                                  num_cores=2, num_subcores=16)
    deg_call = pl.kernel(
        _deg_body,
        out_type=jax.ShapeDtypeStruct((2, NP), jnp.float32),
        mesh=mesh,
        scratch_types=[
            pltpu.VMEM((NCH, CH), jnp.int32),
            pltpu.VMEM((CH,), jnp.float32),
            pltpu.VMEM_SHARED((NP,), jnp.float32),
        ],
    )
    edge_mesh = plsc.VectorSubcoreMesh(core_axis_name="c",
                                       subcore_axis_name="s",
                                       num_cores=1, num_subcores=16)
    edge_call = pl.kernel(
        _edge_body,
        out_type=jax.ShapeDtypeStruct((NP, HID), jnp.float32),
        mesh=edge_mesh,
        scratch_types=[
            pltpu.VMEM((DB, CH), jnp.int32),
            pltpu.VMEM((DB, CH), jnp.int32),
            pltpu.VMEM((DB, CH), jnp.int32),
            pltpu.VMEM((DB, CH), jnp.int32),
            pltpu.VMEM((CH, HID), jnp.float32),
            pltpu.VMEM((CH, HID), jnp.float32),
            pltpu.VMEM_SHARED((NP, HID), jnp.float32),
            pltpu.SemaphoreType.DMA,
            pltpu.SemaphoreType.DMA,
            pltpu.SemaphoreType.DMA,
            pltpu.SemaphoreType.DMA,
            pltpu.SemaphoreType.DMA,
            pltpu.SemaphoreType.DMA,
            pltpu.SemaphoreType.DMA,
            pltpu.SemaphoreType.DMA,
        ],
    )
    return deg_call, edge_call


DB = 8
NCHT = EP // CH
C0N = 160


def _edge_body(g, src2, dst2, zeros2_hbm, out,
               ssl0, ssl1, dsl0, dsl1, rows0, rows1, acc,
               sg0, sg1, ss0, ss1, sd0, sd1, se0, se1):
    s = lax.axis_index("s")
    tn = C0N
    nb = tn // DB
    tbase = s * C0N

    pltpu.sync_copy(zeros2_hbm, rows0)
    for z in range(RPT // CH):
        pltpu.sync_copy(rows0, acc.at[pl.ds(s * RPT + z * CH, CH)])
    pltpu.async_copy(src2.at[pl.ds(tbase, DB)], ssl0, se0)
    pltpu.async_copy(src2.at[pl.ds(tbase + DB, DB)], ssl1, se1)
    pltpu.async_copy(dst2.at[pl.ds(tbase, DB)], dsl0, sd0)
    pltpu.async_copy(dst2.at[pl.ds(tbase + DB, DB)], dsl1, sd1)
    pltpu.make_async_copy(src2.at[pl.ds(tbase, DB)], ssl0, se0).wait()
    pltpu.async_copy(g.at[ssl0.at[0]], rows0, sg0)
    pltpu.async_copy(g.at[ssl0.at[1]], rows1, sg1)
    plsc.subcore_barrier()

    def half(j, ssl_cur, se_cur, ssl_nxt, se_nxt, dsl, sd):
        pltpu.make_async_copy(dst2.at[pl.ds(tbase, DB)], dsl, sd).wait()
        for i in range(DB):
            rows, sg, ss = ((rows0, sg0, ss0) if i % 2 == 0
                            else (rows1, sg1, ss1))
            pltpu.make_async_copy(g.at[ssl_cur.at[0]], rows, sg).wait()
            pltpu.async_copy(rows, acc.at[dsl.at[i]], ss, add=True).wait()
            if i == DB - 2:
                @pl.when(j < nb - 1)
                def _():
                    pltpu.make_async_copy(
                        src2.at[pl.ds(tbase, DB)], ssl_nxt, se_nxt).wait()
            if i < DB - 2:
                pltpu.async_copy(g.at[ssl_cur.at[i + 2]], rows, sg)
            else:
                pltpu.async_copy(g.at[ssl_nxt.at[i - (DB - 2)]], rows, sg)

        @pl.when(j + 2 < nb)
        def _():
            pltpu.async_copy(src2.at[pl.ds(tbase + (j + 2) * DB, DB)],
                             ssl_cur, se_cur)
            pltpu.async_copy(dst2.at[pl.ds(tbase + (j + 2) * DB, DB)],
                             dsl, sd)

    def body(j0, carry):
        half(2 * j0, ssl0, se0, ssl1, se1, dsl0, sd0)
        half(2 * j0 + 1, ssl1, se1, ssl0, se0, dsl1, sd1)
        return carry

    lax.fori_loop(0, nb // 2, body, 0)
    pltpu.make_async_copy(g.at[ssl0.at[0]], rows0, sg0).wait()
    pltpu.make_async_copy(g.at[ssl0.at[0]], rows1, sg1).wait()
    plsc.subcore_barrier()
    for z in range(RPT // CH):
        pltpu.sync_copy(acc.at[pl.ds(s * RPT + z * CH, CH)],
                        out.at[pl.ds(s * RPT + z * CH, CH)])


def _dinv(degsc):
    ones2 = jnp.ones((2, 1), jnp.float32)
    deg = lax.dot_general(degsc, ones2, (((0,), (0,)), ((), ())),
                          preferred_element_type=jnp.float32)
    return lax.rsqrt(deg + 1.0)


def _k1_body(x_ref, w_ref, degsc_ref, g_ref):
    dinv = _dinv(degsc_ref[...])
    g_ref[...] = jnp.dot(x_ref[...], w_ref[...],
                         preferred_element_type=jnp.float32) * dinv


def _gk1(x_p, W1, deg_sc):
    return pl.pallas_call(
        _k1_body,
        grid=(NBLK,),
        in_specs=[
            pl.BlockSpec((BLK, IN_CH), lambda i: (i, 0)),
            pl.BlockSpec((IN_CH, HID), lambda i: (0, 0)),
            pl.BlockSpec((2, BLK), lambda i: (0, i)),
        ],
        out_specs=pl.BlockSpec((BLK, HID), lambda i: (i, 0)),
        out_shape=jax.ShapeDtypeStruct((NP, HID), jnp.float32),
    )(x_p, W1, deg_sc)


def _k2_body(s_ref, g_ref, degsc_ref, b_ref, w_ref, out_ref):
    dinv = _dinv(degsc_ref[...])
    h = dinv * (s_ref[...] + g_ref[...]) + b_ref[...]
    h = jnp.maximum(h, 0.0)
    out_ref[...] = jnp.dot(h, w_ref[...],
                           preferred_element_type=jnp.float32) * dinv


def _gk2(s1, g1, deg_sc, b1r, W2):
    return pl.pallas_call(
        _k2_body,
        grid=(NBLK,),
        in_specs=[
            pl.BlockSpec((BLK, HID), lambda i: (i, 0)),
            pl.BlockSpec((BLK, HID), lambda i: (i, 0)),
            pl.BlockSpec((2, BLK), lambda i: (0, i)),
            pl.BlockSpec((1, HID), lambda i: (0, 0)),
            pl.BlockSpec((HID, HID), lambda i: (0, 0)),
        ],
        out_specs=pl.BlockSpec((BLK, HID), lambda i: (i, 0)),
        out_shape=jax.ShapeDtypeStruct((NP, HID), jnp.float32),
    )(s1, g1, deg_sc, b1r, W2)


def _k3_body(s_ref, g_ref, degsc_ref, b_ref, batch_ref, wl1_ref, bl1_ref,
             wl2_ref, bl2_ref, out_ref, sums_ref, counts_ref):
    i = pl.program_id(0)
    dinv = _dinv(degsc_ref[...])
    h = dinv * (s_ref[...] + g_ref[...]) + b_ref[...]
    h = jnp.maximum(h, 0.0)
    rid = lax.broadcasted_iota(jnp.int32, (BLK, HID), 0) + i * BLK
    h = jnp.where(rid < N_NODES, h, 0.0)
    b2d = batch_ref[0]
    ohT = (lax.broadcasted_iota(jnp.int32, (NUM_GRAPHS, BLK), 0)
           == b2d).astype(jnp.float32)
    sc = lax.dot_general(ohT, h, (((1,), (0,)), ((), ())),
                         preferred_element_type=jnp.float32)
    cc = jnp.sum(ohT, axis=1, keepdims=True)

    @pl.when(i == 0)
    def _():
        sums_ref[...] = jnp.zeros((NUM_GRAPHS, HID), jnp.float32)
        counts_ref[...] = jnp.zeros((NUM_GRAPHS, HID), jnp.float32)

    sums_ref[...] += sc
    counts_ref[...] += cc

    @pl.when(i == NBLK - 1)
    def _():
        counts = counts_ref[:, 0:1]
        pooled = sums_ref[...] / jnp.maximum(counts, 1.0)
        z = jnp.dot(pooled, wl1_ref[...],
                    preferred_element_type=jnp.float32) + bl1_ref[...]
        z = jnp.maximum(z, 0.0)
        logits = jnp.dot(z, wl2_ref[...],
                         preferred_element_type=jnp.float32) + bl2_ref[...]
        m = jnp.max(logits, axis=1, keepdims=True)
        e = logits - m
        out_ref[...] = e - jnp.log(jnp.sum(jnp.exp(e), axis=1, keepdims=True))


def _gk3(s2, g2, deg_sc, b2r, batch3, Wl1, bl1r, Wl2, bl2r):
    return pl.pallas_call(
        _k3_body,
        grid=(NBLK,),
        in_specs=[
            pl.BlockSpec((BLK, HID), lambda i: (i, 0)),
            pl.BlockSpec((BLK, HID), lambda i: (i, 0)),
            pl.BlockSpec((2, BLK), lambda i: (0, i)),
            pl.BlockSpec((1, HID), lambda i: (0, 0)),
            pl.BlockSpec((1, 1, BLK), lambda i: (i, 0, 0)),
            pl.BlockSpec((HID, 64), lambda i: (0, 0)),
            pl.BlockSpec((1, 64), lambda i: (0, 0)),
            pl.BlockSpec((64, NUM_CLASSES), lambda i: (0, 0)),
            pl.BlockSpec((1, NUM_CLASSES), lambda i: (0, 0)),
        ],
        out_specs=pl.BlockSpec((NUM_GRAPHS, NUM_CLASSES), lambda i: (0, 0)),
        out_shape=jax.ShapeDtypeStruct((NUM_GRAPHS, NUM_CLASSES), jnp.float32),
        scratch_shapes=[
            pltpu.VMEM((NUM_GRAPHS, HID), jnp.float32),
            pltpu.VMEM((NUM_GRAPHS, HID), jnp.float32),
        ],
    )(s2, g2, deg_sc, b2r, batch3, Wl1, bl1r, Wl2, bl2r)


def kernel(x, edge_index, batch, W1, b1, W2, b2, Wl1, bl1, Wl2, bl2):
    src = edge_index[0].astype(jnp.int32)
    dst = edge_index[1].astype(jnp.int32)
    pad = jnp.full((EP - E,), N_NODES, jnp.int32)
    src2 = jnp.concatenate([src, pad]).reshape(NCHT, CH)
    dst2 = jnp.concatenate([dst, pad]).reshape(NCHT, CH)
    x_p = jnp.concatenate(
        [x, jnp.zeros((NP - N_NODES, IN_CH), jnp.float32)], axis=0)
    batch_p = jnp.concatenate(
        [batch.astype(jnp.int32),
         jnp.full((NP - N_NODES,), NUM_GRAPHS, jnp.int32)])
    batch3 = batch_p.reshape(NBLK, 1, BLK)
    b1r = b1.reshape(1, HID)
    b2r = b2.reshape(1, HID)
    bl1r = bl1.reshape(1, 64)
    bl2r = bl2.reshape(1, NUM_CLASSES)
    ones_ch = jnp.ones((CH,), jnp.float32)
    zeros1 = jnp.zeros((RPT,), jnp.float32)
    zeros2 = jnp.zeros((CH, HID), jnp.float32)

    _deg_call, _edge_call = _sc_calls()
    deg_sc = _deg_call(dst2, ones_ch, zeros1)
    g1 = _gk1(x_p, W1, deg_sc)
    s1 = _edge_call(g1, src2, dst2, zeros2)
    g2 = _gk2(s1, g1, deg_sc, b1r, W2)
    s2 = _edge_call(g2, src2, dst2, zeros2)
    return _gk3(s2, g2, deg_sc, b2r, batch3, Wl1, bl1r, Wl2, bl2r)


# speedup vs baseline: 1.3330x; 1.3330x over previous
import jax
import jax.numpy as jnp
from jax import lax
from jax.experimental import pallas as pl
from jax.experimental.pallas import tpu as pltpu
from jax.experimental.pallas import tpu_sc as plsc

N_NODES = 10000
IN_CH = 128
HID = 128
NUM_CLASSES = 10
NUM_GRAPHS = 128

NP = 10240
E = 320000
CH = 128
NCH = 80
NTILES = 32
EP = NTILES * NCH * CH
RPT = NP // 16
BLK = 1024
NBLK = NP // BLK


def _deg_body(dst2, ones_hbm, zeros1_hbm, out, didx, ones_v, acc1):
    c = lax.axis_index("c")
    s = lax.axis_index("s")
    tid = s * 2 + c
    pltpu.sync_copy(dst2.at[pl.ds(tid * NCH, NCH)], didx)
    pltpu.sync_copy(ones_hbm, ones_v)
    pltpu.sync_copy(zeros1_hbm, acc1.at[pl.ds(s * RPT, RPT)])
    plsc.subcore_barrier()

    def body(k, carry):
        pltpu.sync_copy(ones_v, acc1.at[didx.at[k]], add=True)
        return carry

    lax.fori_loop(0, NCH, body, 0)
    plsc.subcore_barrier()
    pltpu.sync_copy(acc1.at[pl.ds(s * RPT, RPT)], out.at[c, pl.ds(s * RPT, RPT)])


def _sc_calls():
    mesh = plsc.VectorSubcoreMesh(core_axis_name="c", subcore_axis_name="s",
                                  num_cores=2, num_subcores=16)
    deg_call = pl.kernel(
        _deg_body,
        out_type=jax.ShapeDtypeStruct((2, NP), jnp.float32),
        mesh=mesh,
        scratch_types=[
            pltpu.VMEM((NCH, CH), jnp.int32),
            pltpu.VMEM((CH,), jnp.float32),
            pltpu.VMEM_SHARED((NP,), jnp.float32),
        ],
    )
    edge_call = pl.kernel(
        _edge_body,
        out_type=jax.ShapeDtypeStruct((2, NP, HID), jnp.float32),
        mesh=mesh,
        scratch_types=[
            pltpu.VMEM((DB, CH), jnp.int32),
            pltpu.VMEM((DB, CH), jnp.int32),
            pltpu.VMEM((DB, CH), jnp.int32),
            pltpu.VMEM((DB, CH), jnp.int32),
            pltpu.VMEM((CH, HID), jnp.float32),
            pltpu.VMEM((CH, HID), jnp.float32),
            pltpu.VMEM_SHARED((NP, HID), jnp.float32),
            pltpu.SemaphoreType.DMA,
            pltpu.SemaphoreType.DMA,
            pltpu.SemaphoreType.DMA,
            pltpu.SemaphoreType.DMA,
            pltpu.SemaphoreType.DMA,
            pltpu.SemaphoreType.DMA,
            pltpu.SemaphoreType.DMA,
            pltpu.SemaphoreType.DMA,
        ],
    )
    return deg_call, edge_call


DB = 8
NCHT = EP // CH
C0N = 112
C1N = (NCHT // 16) - C0N


def _edge_body(g, src2, dst2, zeros2_hbm, out,
               ssl0, ssl1, dsl0, dsl1, rows0, rows1, acc,
               sg0, sg1, ss0, ss1, sd0, sd1, se0, se1):
    c = lax.axis_index("c")
    s = lax.axis_index("s")
    tn = jnp.where(c == 0, C0N, C1N)
    nb = tn // DB
    tbase = jnp.where(c == 0, s * C0N, 16 * C0N + s * C1N)

    pltpu.sync_copy(zeros2_hbm, rows0)
    for z in range(RPT // CH):
        pltpu.sync_copy(rows0, acc.at[pl.ds(s * RPT + z * CH, CH)])
    pltpu.async_copy(src2.at[pl.ds(tbase, DB)], ssl0, se0)
    pltpu.async_copy(src2.at[pl.ds(tbase + DB, DB)], ssl1, se1)
    pltpu.async_copy(dst2.at[pl.ds(tbase, DB)], dsl0, sd0)
    pltpu.async_copy(dst2.at[pl.ds(tbase + DB, DB)], dsl1, sd1)
    pltpu.make_async_copy(src2.at[pl.ds(tbase, DB)], ssl0, se0).wait()
    pltpu.async_copy(g.at[ssl0.at[0]], rows0, sg0)
    pltpu.async_copy(g.at[ssl0.at[1]], rows1, sg1)
    plsc.subcore_barrier()

    def half(j, ssl_cur, se_cur, ssl_nxt, se_nxt, dsl, sd):
        pltpu.make_async_copy(dst2.at[pl.ds(tbase, DB)], dsl, sd).wait()
        for i in range(DB):
            rows, sg, ss = ((rows0, sg0, ss0) if i % 2 == 0
                            else (rows1, sg1, ss1))
            pltpu.make_async_copy(g.at[ssl_cur.at[0]], rows, sg).wait()
            pltpu.async_copy(rows, acc.at[dsl.at[i]], ss, add=True).wait()
            if i == DB - 2:
                @pl.when(j < nb - 1)
                def _():
                    pltpu.make_async_copy(
                        src2.at[pl.ds(tbase, DB)], ssl_nxt, se_nxt).wait()
            if i < DB - 2:
                pltpu.async_copy(g.at[ssl_cur.at[i + 2]], rows, sg)
            else:
                pltpu.async_copy(g.at[ssl_nxt.at[i - (DB - 2)]], rows, sg)

        @pl.when(j + 2 < nb)
        def _():
            pltpu.async_copy(src2.at[pl.ds(tbase + (j + 2) * DB, DB)],
                             ssl_cur, se_cur)
            pltpu.async_copy(dst2.at[pl.ds(tbase + (j + 2) * DB, DB)],
                             dsl, sd)

    def body(j0, carry):
        half(2 * j0, ssl0, se0, ssl1, se1, dsl0, sd0)
        half(2 * j0 + 1, ssl1, se1, ssl0, se0, dsl1, sd1)
        return carry

    lax.fori_loop(0, nb // 2, body, 0)
    pltpu.make_async_copy(g.at[ssl0.at[0]], rows0, sg0).wait()
    pltpu.make_async_copy(g.at[ssl0.at[0]], rows1, sg1).wait()
    plsc.subcore_barrier()
    for z in range(RPT // CH):
        pltpu.sync_copy(acc.at[pl.ds(s * RPT + z * CH, CH)],
                        out.at[c, pl.ds(s * RPT + z * CH, CH)])


def _dinv(degsc):
    ones2 = jnp.ones((2, 1), jnp.float32)
    deg = lax.dot_general(degsc, ones2, (((0,), (0,)), ((), ())),
                          preferred_element_type=jnp.float32)
    return lax.rsqrt(deg + 1.0)


def _k1_body(x_ref, w_ref, degsc_ref, g_ref):
    dinv = _dinv(degsc_ref[...])
    g_ref[...] = jnp.dot(x_ref[...], w_ref[...],
                         preferred_element_type=jnp.float32) * dinv


def _gk1(x_p, W1, deg_sc):
    return pl.pallas_call(
        _k1_body,
        grid=(NBLK,),
        in_specs=[
            pl.BlockSpec((BLK, IN_CH), lambda i: (i, 0)),
            pl.BlockSpec((IN_CH, HID), lambda i: (0, 0)),
            pl.BlockSpec((2, BLK), lambda i: (0, i)),
        ],
        out_specs=pl.BlockSpec((BLK, HID), lambda i: (i, 0)),
        out_shape=jax.ShapeDtypeStruct((NP, HID), jnp.float32),
    )(x_p, W1, deg_sc)


def _k2_body(s_ref, g_ref, degsc_ref, b_ref, w_ref, out_ref):
    dinv = _dinv(degsc_ref[...])
    h = dinv * (s_ref[0] + s_ref[1] + g_ref[...]) + b_ref[...]
    h = jnp.maximum(h, 0.0)
    out_ref[...] = jnp.dot(h, w_ref[...],
                           preferred_element_type=jnp.float32) * dinv


def _gk2(s1, g1, deg_sc, b1r, W2):
    return pl.pallas_call(
        _k2_body,
        grid=(NBLK,),
        in_specs=[
            pl.BlockSpec((2, BLK, HID), lambda i: (0, i, 0)),
            pl.BlockSpec((BLK, HID), lambda i: (i, 0)),
            pl.BlockSpec((2, BLK), lambda i: (0, i)),
            pl.BlockSpec((1, HID), lambda i: (0, 0)),
            pl.BlockSpec((HID, HID), lambda i: (0, 0)),
        ],
        out_specs=pl.BlockSpec((BLK, HID), lambda i: (i, 0)),
        out_shape=jax.ShapeDtypeStruct((NP, HID), jnp.float32),
    )(s1, g1, deg_sc, b1r, W2)


def _k3_body(s_ref, g_ref, degsc_ref, b_ref, batch_ref, wl1_ref, bl1_ref,
             wl2_ref, bl2_ref, out_ref, sums_ref, counts_ref):
    i = pl.program_id(0)
    dinv = _dinv(degsc_ref[...])
    h = dinv * (s_ref[0] + s_ref[1] + g_ref[...]) + b_ref[...]
    h = jnp.maximum(h, 0.0)
    rid = lax.broadcasted_iota(jnp.int32, (BLK, HID), 0) + i * BLK
    h = jnp.where(rid < N_NODES, h, 0.0)
    b2d = batch_ref[0]
    ohT = (lax.broadcasted_iota(jnp.int32, (NUM_GRAPHS, BLK), 0)
           == b2d).astype(jnp.float32)
    sc = lax.dot_general(ohT, h, (((1,), (0,)), ((), ())),
                         preferred_element_type=jnp.float32)
    cc = jnp.sum(ohT, axis=1, keepdims=True)

    @pl.when(i == 0)
    def _():
        sums_ref[...] = jnp.zeros((NUM_GRAPHS, HID), jnp.float32)
        counts_ref[...] = jnp.zeros((NUM_GRAPHS, HID), jnp.float32)

    sums_ref[...] += sc
    counts_ref[...] += cc

    @pl.when(i == NBLK - 1)
    def _():
        counts = counts_ref[:, 0:1]
        pooled = sums_ref[...] / jnp.maximum(counts, 1.0)
        z = jnp.dot(pooled, wl1_ref[...],
                    preferred_element_type=jnp.float32) + bl1_ref[...]
        z = jnp.maximum(z, 0.0)
        logits = jnp.dot(z, wl2_ref[...],
                         preferred_element_type=jnp.float32) + bl2_ref[...]
        m = jnp.max(logits, axis=1, keepdims=True)
        e = logits - m
        out_ref[...] = e - jnp.log(jnp.sum(jnp.exp(e), axis=1, keepdims=True))


def _gk3(s2, g2, deg_sc, b2r, batch3, Wl1, bl1r, Wl2, bl2r):
    return pl.pallas_call(
        _k3_body,
        grid=(NBLK,),
        in_specs=[
            pl.BlockSpec((2, BLK, HID), lambda i: (0, i, 0)),
            pl.BlockSpec((BLK, HID), lambda i: (i, 0)),
            pl.BlockSpec((2, BLK), lambda i: (0, i)),
            pl.BlockSpec((1, HID), lambda i: (0, 0)),
            pl.BlockSpec((1, 1, BLK), lambda i: (i, 0, 0)),
            pl.BlockSpec((HID, 64), lambda i: (0, 0)),
            pl.BlockSpec((1, 64), lambda i: (0, 0)),
            pl.BlockSpec((64, NUM_CLASSES), lambda i: (0, 0)),
            pl.BlockSpec((1, NUM_CLASSES), lambda i: (0, 0)),
        ],
        out_specs=pl.BlockSpec((NUM_GRAPHS, NUM_CLASSES), lambda i: (0, 0)),
        out_shape=jax.ShapeDtypeStruct((NUM_GRAPHS, NUM_CLASSES), jnp.float32),
        scratch_shapes=[
            pltpu.VMEM((NUM_GRAPHS, HID), jnp.float32),
            pltpu.VMEM((NUM_GRAPHS, HID), jnp.float32),
        ],
    )(s2, g2, deg_sc, b2r, batch3, Wl1, bl1r, Wl2, bl2r)


def kernel(x, edge_index, batch, W1, b1, W2, b2, Wl1, bl1, Wl2, bl2):
    src = edge_index[0].astype(jnp.int32)
    dst = edge_index[1].astype(jnp.int32)
    pad = jnp.full((EP - E,), N_NODES, jnp.int32)
    src2 = jnp.concatenate([src, pad]).reshape(NCHT, CH)
    dst2 = jnp.concatenate([dst, pad]).reshape(NCHT, CH)
    x_p = jnp.concatenate(
        [x, jnp.zeros((NP - N_NODES, IN_CH), jnp.float32)], axis=0)
    batch_p = jnp.concatenate(
        [batch.astype(jnp.int32),
         jnp.full((NP - N_NODES,), NUM_GRAPHS, jnp.int32)])
    batch3 = batch_p.reshape(NBLK, 1, BLK)
    b1r = b1.reshape(1, HID)
    b2r = b2.reshape(1, HID)
    bl1r = bl1.reshape(1, 64)
    bl2r = bl2.reshape(1, NUM_CLASSES)
    ones_ch = jnp.ones((CH,), jnp.float32)
    zeros1 = jnp.zeros((RPT,), jnp.float32)
    zeros2 = jnp.zeros((CH, HID), jnp.float32)

    _deg_call, _edge_call = _sc_calls()
    deg_sc = _deg_call(dst2, ones_ch, zeros1)
    g1 = _gk1(x_p, W1, deg_sc)
    s1 = _edge_call(g1, src2, dst2, zeros2)
    g2 = _gk2(s1, g1, deg_sc, b1r, W2)
    s2 = _edge_call(g2, src2, dst2, zeros2)
    return _gk3(s2, g2, deg_sc, b2r, batch3, Wl1, bl1r, Wl2, bl2r)
